# Initial kernel scaffold; baseline (speedup 1.0000x reference)
import functools

import jax
import jax.numpy as jnp
from jax import lax
from jax.experimental import pallas as pl
from jax.experimental.pallas import tpu as pltpu
from jax.experimental.pallas import tpu_sc as plsc

_B, _D, _NB = 16384, 128, 100
_EPS = 1e-06
_NC, _NS, _L = 2, 16, 16
_NW = _NC * _NS
_RW = _B // _NW
_SUB = 128
_NSUB = _RW // _SUB


def _prep_body(bc_ref, rm_ref, rv_ref, sm_ref, sv_ref, scale_ref, shift_ref):
    early = jnp.sum(bc_ref[...]) < jnp.float32(_NB * 0.1)
    scale = (jnp.sqrt(sv_ref[...]) + _EPS) / (jnp.sqrt(rv_ref[...]) + _EPS)
    shift = sm_ref[...] - rm_ref[...] * scale
    scale_ref[...] = jnp.where(early, jnp.float32(1.0), scale)
    shift_ref[...] = jnp.where(early, jnp.float32(0.0), shift)


def _prep(rm, rv, sm, sv, bc):
    return pl.pallas_call(
        _prep_body,
        out_shape=(
            jax.ShapeDtypeStruct((_NB, _D), jnp.float32),
            jax.ShapeDtypeStruct((_NB, _D), jnp.float32),
        ),
    )(bc.reshape(1, _NB), rm, rv, sm, sv)


def _sc_smooth(features, targets, scale, shift):
    mesh = plsc.VectorSubcoreMesh(core_axis_name="c", subcore_axis_name="s")

    @functools.partial(
        pl.kernel,
        mesh=mesh,
        out_type=jax.ShapeDtypeStruct((_B, _D), jnp.float32),
        scratch_types=[
            pltpu.VMEM((_NB, _D), jnp.float32),
            pltpu.VMEM((_NB, _D), jnp.float32),
            pltpu.VMEM((_RW,), jnp.float32),
            pltpu.VMEM((_RW,), jnp.int32),
            pltpu.VMEM((_SUB, _D), jnp.float32),
            pltpu.VMEM((_SUB, _D), jnp.float32),
            pltpu.VMEM((_SUB, _D), jnp.float32),
            pltpu.VMEM((_SUB, _D), jnp.float32),
            pltpu.SemaphoreType.DMA,
            pltpu.SemaphoreType.DMA,
            pltpu.SemaphoreType.DMA,
            pltpu.SemaphoreType.DMA,
            pltpu.SemaphoreType.DMA,
        ],
    )
    def k(feat_hbm, tgt_hbm, scale_hbm, shift_hbm, out_hbm,
          scale_v, shift_v, tgt_v, bins_v, fin0, fin1, fout0, fout1,
          sem_t, sem_i0, sem_i1, sem_o0, sem_o1):
        wid = lax.axis_index("s") * _NC + lax.axis_index("c")
        base = wid * _RW

        pltpu.make_async_copy(scale_hbm, scale_v, sem_t).start()
        pltpu.make_async_copy(shift_hbm, shift_v, sem_t).start()
        pltpu.sync_copy(tgt_hbm.at[pl.ds(base, _RW)], tgt_v)

        def binbody(i, carry):
            t = tgt_v[pl.ds(i * _L, _L)]
            bi = (t * jnp.float32(_NB)).astype(jnp.int32)
            bins_v[pl.ds(i * _L, _L)] = jnp.clip(bi, 0, _NB - 1)
            return carry

        lax.fori_loop(0, _RW // _L, binbody, 0)

        pltpu.make_async_copy(scale_hbm, scale_v, sem_t).wait()
        pltpu.make_async_copy(shift_hbm, shift_v, sem_t).wait()

        fins = (fin0, fin1)
        fouts = (fout0, fout1)
        sis = (sem_i0, sem_i1)
        sos = (sem_o0, sem_o1)

        def in_cp(ci):
            return pltpu.make_async_copy(
                feat_hbm.at[pl.ds(base + ci * _SUB, _SUB)], fins[ci % 2],
                sis[ci % 2])

        def out_cp(ci):
            return pltpu.make_async_copy(
                fouts[ci % 2], out_hbm.at[pl.ds(base + ci * _SUB, _SUB)],
                sos[ci % 2])

        in_cp(0).start()
        in_cp(1).start()

        lane = lax.iota(jnp.int32, _L)

        for ci in range(_NSUB):
            in_cp(ci).wait()
            if ci >= 2:
                out_cp(ci - 2).wait()
            fi = fins[ci % 2]
            fo = fouts[ci % 2]

            def rowbody(r, carry, fi=fi, fo=fo):
                splat = jnp.full((_L,), 0, jnp.int32) + r
                binsplat = plsc.load_gather(bins_v, [splat])
                for j in range(_D // _L):
                    col = lane + (j * _L)
                    sc = plsc.load_gather(scale_v, [binsplat, col])
                    sh = plsc.load_gather(shift_v, [binsplat, col])
                    f = fi[r, pl.ds(j * _L, _L)]
                    fo[r, pl.ds(j * _L, _L)] = f * sc + sh
                return carry

            lax.fori_loop(0, _SUB, rowbody, 0)
            out_cp(ci).start()
            if ci + 2 < _NSUB:
                in_cp(ci + 2).start()

        out_cp(_NSUB - 2).wait()
        out_cp(_NSUB - 1).wait()

    return k(features, targets, scale, shift)


@jax.jit
def _impl(features, targets, running_mean, running_var,
          smoothed_mean, smoothed_var, bin_counts):
    scale, shift = _prep(running_mean, running_var,
                         smoothed_mean, smoothed_var, bin_counts)
    return _sc_smooth(features, targets, scale, shift)


def kernel(features, targets, running_mean, running_var,
           smoothed_mean, smoothed_var, bin_counts):
    return _impl(features, targets, running_mean, running_var,
                 smoothed_mean, smoothed_var, bin_counts)


# speedup vs baseline: 4.1727x; 4.1727x over previous
import functools

import jax
import jax.numpy as jnp
from jax import lax
from jax.experimental import pallas as pl
from jax.experimental.pallas import tpu as pltpu
from jax.experimental.pallas import tpu_sc as plsc

_B, _D, _NB = 16384, 128, 100
_EPS = 1e-06
_NC, _NS, _L = 2, 16, 16
_NW = _NC * _NS
_RW = _B // _NW
_SUB = 128
_NSUB = _RW // _SUB


def _prep_body(bc_ref, rm_ref, rv_ref, sm_ref, sv_ref, scale_ref, shift_ref):
    early = jnp.sum(bc_ref[...]) < jnp.float32(_NB * 0.1)
    scale = (jnp.sqrt(sv_ref[...]) + _EPS) / (jnp.sqrt(rv_ref[...]) + _EPS)
    shift = sm_ref[...] - rm_ref[...] * scale
    scale_ref[...] = jnp.where(early, jnp.float32(1.0), scale)
    shift_ref[...] = jnp.where(early, jnp.float32(0.0), shift)


def _prep(rm, rv, sm, sv, bc):
    return pl.pallas_call(
        _prep_body,
        out_shape=(
            jax.ShapeDtypeStruct((_NB, _D), jnp.float32),
            jax.ShapeDtypeStruct((_NB, _D), jnp.float32),
        ),
    )(bc.reshape(1, _NB), rm, rv, sm, sv)


def _sc_smooth(features, targets, scale, shift):
    mesh = plsc.VectorSubcoreMesh(core_axis_name="c", subcore_axis_name="s")

    @functools.partial(
        pl.kernel,
        mesh=mesh,
        compiler_params=pltpu.CompilerParams(needs_layout_passes=False),
        out_type=jax.ShapeDtypeStruct((_B, _D), jnp.float32),
        scratch_types=[
            pltpu.VMEM((_NB, _D), jnp.float32),
            pltpu.VMEM((_NB, _D), jnp.float32),
            pltpu.VMEM((_RW,), jnp.float32),
            pltpu.VMEM((_RW,), jnp.int32),
            pltpu.VMEM((_SUB, _D), jnp.float32),
            pltpu.VMEM((_SUB, _D), jnp.float32),
            pltpu.VMEM((_SUB, _D), jnp.float32),
            pltpu.VMEM((_SUB, _D), jnp.float32),
            pltpu.SemaphoreType.DMA,
            pltpu.SemaphoreType.DMA,
            pltpu.SemaphoreType.DMA,
            pltpu.SemaphoreType.DMA,
            pltpu.SemaphoreType.DMA,
        ],
    )
    def k(feat_hbm, tgt_hbm, scale_hbm, shift_hbm, out_hbm,
          scale_v, shift_v, tgt_v, bins_v, fin0, fin1, fout0, fout1,
          sem_t, sem_i0, sem_i1, sem_o0, sem_o1):
        wid = lax.axis_index("s") * _NC + lax.axis_index("c")
        base = wid * _RW

        pltpu.make_async_copy(scale_hbm, scale_v, sem_t).start()
        pltpu.make_async_copy(shift_hbm, shift_v, sem_t).start()
        pltpu.sync_copy(tgt_hbm.at[pl.ds(base, _RW)], tgt_v)

        def binbody(i, carry):
            t = tgt_v[pl.ds(i * _L, _L)]
            bi = (t * jnp.float32(_NB)).astype(jnp.int32)
            bins_v[pl.ds(i * _L, _L)] = jnp.clip(bi, 0, _NB - 1)
            return carry

        lax.fori_loop(0, _RW // _L, binbody, 0)

        pltpu.make_async_copy(scale_hbm, scale_v, sem_t).wait()
        pltpu.make_async_copy(shift_hbm, shift_v, sem_t).wait()

        fins = (fin0, fin1)
        fouts = (fout0, fout1)
        sis = (sem_i0, sem_i1)
        sos = (sem_o0, sem_o1)

        def in_cp(ci):
            return pltpu.make_async_copy(
                feat_hbm.at[pl.ds(base + ci * _SUB, _SUB)], fins[ci % 2],
                sis[ci % 2])

        def out_cp(ci):
            return pltpu.make_async_copy(
                fouts[ci % 2], out_hbm.at[pl.ds(base + ci * _SUB, _SUB)],
                sos[ci % 2])

        in_cp(0).start()
        in_cp(1).start()

        lane = lax.iota(jnp.int32, _L)

        for ci in range(_NSUB):
            in_cp(ci).wait()
            if ci >= 2:
                out_cp(ci - 2).wait()
            fi = fins[ci % 2]
            fo = fouts[ci % 2]

            def rowbody(r, carry, fi=fi, fo=fo, ci=ci):
                splat = jnp.full((_L,), ci * _SUB, jnp.int32) + r
                binsplat = plsc.load_gather(bins_v, [splat])
                for j in range(_D // _L):
                    col = lane + (j * _L)
                    sc = plsc.load_gather(scale_v, [binsplat, col])
                    sh = plsc.load_gather(shift_v, [binsplat, col])
                    f = fi[r, pl.ds(j * _L, _L)]
                    fo[r, pl.ds(j * _L, _L)] = f * sc + sh
                return carry

            lax.fori_loop(0, _SUB, rowbody, 0)
            out_cp(ci).start()
            if ci + 2 < _NSUB:
                in_cp(ci + 2).start()

        out_cp(_NSUB - 2).wait()
        out_cp(_NSUB - 1).wait()

    return k(features, targets, scale, shift)


@jax.jit
def _impl(features, targets, running_mean, running_var,
          smoothed_mean, smoothed_var, bin_counts):
    scale, shift = _prep(running_mean, running_var,
                         smoothed_mean, smoothed_var, bin_counts)
    return _sc_smooth(features, targets, scale, shift)


def kernel(features, targets, running_mean, running_var,
           smoothed_mean, smoothed_var, bin_counts):
    return _impl(features, targets, running_mean, running_var,
                 smoothed_mean, smoothed_var, bin_counts)


# speedup vs baseline: 4.2068x; 1.0082x over previous
import functools

import jax
import jax.numpy as jnp
from jax import lax
from jax.experimental import pallas as pl
from jax.experimental.pallas import tpu as pltpu
from jax.experimental.pallas import tpu_sc as plsc

_B, _D, _NB = 16384, 128, 100
_EPS = 1e-06
_NC, _NS, _L = 2, 16, 16
_NW = _NC * _NS
_RW = _B // _NW
_SUB = 128
_NSUB = _RW // _SUB


def _prep_body(bc_ref, rm_ref, rv_ref, sm_ref, sv_ref, scale_ref, shift_ref):
    early = jnp.sum(bc_ref[...]) < jnp.float32(_NB * 0.1)
    scale = (jnp.sqrt(sv_ref[...]) + _EPS) / (jnp.sqrt(rv_ref[...]) + _EPS)
    shift = sm_ref[...] - rm_ref[...] * scale
    scale_ref[...] = jnp.where(early, jnp.float32(1.0), scale)
    shift_ref[...] = jnp.where(early, jnp.float32(0.0), shift)


def _prep(rm, rv, sm, sv, bc):
    return pl.pallas_call(
        _prep_body,
        out_shape=(
            jax.ShapeDtypeStruct((_NB, _D), jnp.float32),
            jax.ShapeDtypeStruct((_NB, _D), jnp.float32),
        ),
    )(bc.reshape(1, _NB), rm, rv, sm, sv)


def _sc_smooth(features, targets, scale, shift):
    mesh = plsc.VectorSubcoreMesh(core_axis_name="c", subcore_axis_name="s")

    @functools.partial(
        pl.kernel,
        mesh=mesh,
        compiler_params=pltpu.CompilerParams(needs_layout_passes=False),
        out_type=jax.ShapeDtypeStruct((_B, _D), jnp.float32),
        scratch_types=[
            pltpu.VMEM((_NB, _D), jnp.float32),
            pltpu.VMEM((_NB, _D), jnp.float32),
            pltpu.VMEM((_RW,), jnp.float32),
            pltpu.VMEM((_RW,), jnp.int32),
            pltpu.VMEM((_SUB, _D), jnp.float32),
            pltpu.VMEM((_SUB, _D), jnp.float32),
            pltpu.VMEM((_SUB, _D), jnp.float32),
            pltpu.VMEM((_SUB, _D), jnp.float32),
            pltpu.SemaphoreType.DMA,
            pltpu.SemaphoreType.DMA,
            pltpu.SemaphoreType.DMA,
            pltpu.SemaphoreType.DMA,
            pltpu.SemaphoreType.DMA,
        ],
    )
    def k(feat_hbm, tgt_hbm, scale_hbm, shift_hbm, out_hbm,
          scale_v, shift_v, tgt_v, bins_v, fin0, fin1, fout0, fout1,
          sem_t, sem_i0, sem_i1, sem_o0, sem_o1):
        wid = lax.axis_index("s") * _NC + lax.axis_index("c")
        base = wid * _RW

        pltpu.make_async_copy(scale_hbm, scale_v, sem_t).start()
        pltpu.make_async_copy(shift_hbm, shift_v, sem_t).start()
        pltpu.sync_copy(tgt_hbm.at[pl.ds(base, _RW)], tgt_v)

        def binbody(i, carry):
            t = tgt_v[pl.ds(i * _L, _L)]
            bi = (t * jnp.float32(_NB)).astype(jnp.int32)
            bins_v[pl.ds(i * _L, _L)] = jnp.clip(bi, 0, _NB - 1)
            return carry

        lax.fori_loop(0, _RW // _L, binbody, 0)

        pltpu.make_async_copy(scale_hbm, scale_v, sem_t).wait()
        pltpu.make_async_copy(shift_hbm, shift_v, sem_t).wait()

        fins = (fin0, fin1)
        fouts = (fout0, fout1)
        sis = (sem_i0, sem_i1)
        sos = (sem_o0, sem_o1)

        def in_cp(ci):
            return pltpu.make_async_copy(
                feat_hbm.at[pl.ds(base + ci * _SUB, _SUB)], fins[ci % 2],
                sis[ci % 2])

        def out_cp(ci):
            return pltpu.make_async_copy(
                fouts[ci % 2], out_hbm.at[pl.ds(base + ci * _SUB, _SUB)],
                sos[ci % 2])

        in_cp(0).start()
        in_cp(1).start()

        for ci in range(_NSUB):
            in_cp(ci).wait()
            if ci >= 2:
                out_cp(ci - 2).wait()
            fi = fins[ci % 2]
            fo = fouts[ci % 2]

            def grpbody(g, carry, fi=fi, fo=fo, ci=ci):
                binv = bins_v[pl.ds(ci * _SUB + g * _L, _L)]
                for l in range(_L):
                    b = binv[l]
                    r = g * _L + l
                    for j in range(_D // _L):
                        sc = scale_v[b, pl.ds(j * _L, _L)]
                        sh = shift_v[b, pl.ds(j * _L, _L)]
                        f = fi[r, pl.ds(j * _L, _L)]
                        fo[r, pl.ds(j * _L, _L)] = f * sc + sh
                return carry

            lax.fori_loop(0, _SUB // _L, grpbody, 0)
            out_cp(ci).start()
            if ci + 2 < _NSUB:
                in_cp(ci + 2).start()

        out_cp(_NSUB - 2).wait()
        out_cp(_NSUB - 1).wait()

    return k(features, targets, scale, shift)


@jax.jit
def _impl(features, targets, running_mean, running_var,
          smoothed_mean, smoothed_var, bin_counts):
    scale, shift = _prep(running_mean, running_var,
                         smoothed_mean, smoothed_var, bin_counts)
    return _sc_smooth(features, targets, scale, shift)


def kernel(features, targets, running_mean, running_var,
           smoothed_mean, smoothed_var, bin_counts):
    return _impl(features, targets, running_mean, running_var,
                 smoothed_mean, smoothed_var, bin_counts)
